# baseline (speedup 1.0000x reference)
import functools

import jax
import jax.numpy as jnp
from jax import lax
from jax.experimental import pallas as pl
from jax.experimental.pallas import tpu as pltpu
from jax.experimental.pallas import tpu_sc as plsc

_NC = 2
_NS = 16
_NW = _NC * _NS
_L = 16

_CHUNK = 64


@functools.partial(jax.jit, static_argnames=("B", "F", "V", "D"))
def _fm_sc(catT, embT, lr_flat, bias, *, B, F, V, D):
    rows_per_w = B // _NW
    n_chunks = rows_per_w // _CHUNK
    idx_len = _CHUNK * F
    n_irows = idx_len // 128

    mesh = plsc.VectorSubcoreMesh(core_axis_name="c", subcore_axis_name="s")

    @functools.partial(
        pl.kernel,
        out_type=jax.ShapeDtypeStruct((B,), jnp.float32),
        mesh=mesh,
        compiler_params=pltpu.CompilerParams(needs_layout_passes=False,
                                             use_tc_tiling_on_sc=False),
        scratch_types=[
            pltpu.VMEM((n_irows, 128), jnp.int32),
            pltpu.VMEM((D * idx_len,), jnp.float32),
            pltpu.VMEM((idx_len,), jnp.float32),
            pltpu.VMEM((rows_per_w,), jnp.float32),
            pltpu.SemaphoreType.DMA,
        ],
    )
    def fm_kernel(catT_hbm, embT_hbm, lr_hbm, out_hbm,
                  idx_v, gbuf, lbuf, out_v, sem):
        wid = lax.axis_index("s") * _NC + lax.axis_index("c")
        row0 = wid * rows_per_w

        zeros16 = jnp.zeros((_L,), jnp.float32)

        def chunk_body(c, carry):
            i0 = pl.multiple_of(row0 + c * _CHUNK, _CHUNK)
            for f in range(F):
                pltpu.sync_copy(
                    catT_hbm.at[f, pl.ds(i0, _CHUNK)],
                    idx_v.at[f // 2, pl.ds((f % 2) * _CHUNK, _CHUNK)])
            for r in range(n_irows):
                for g in range(128 // _L):
                    f = (r * 128 + g * _L) // _CHUNK
                    sl = pl.ds(g * _L, _L)
                    idx_v[r, sl] = idx_v[r, sl] + (f * V)

            copies = []
            for d in range(D):
                for r in range(n_irows):
                    copies.append(pltpu.async_copy(
                        embT_hbm.at[d].at[idx_v.at[r]],
                        gbuf.at[pl.ds((d * n_irows + r) * 128, 128)], sem))
            for r in range(n_irows):
                copies.append(pltpu.async_copy(
                    lr_hbm.at[idx_v.at[r]], lbuf.at[pl.ds(r * 128, 128)], sem))
            for cp in copies:
                cp.wait()

            for g in range(_CHUNK // _L):
                def d_body(d, acc):
                    ss, q = acc
                    dbase = pl.multiple_of(d * idx_len, _L)
                    t = zeros16
                    for f in range(F):
                        e = gbuf[pl.ds(dbase + (f * _CHUNK + g * _L), _L)]
                        t = t + e
                        q = q + e * e
                    return ss + t * t, q

                ss, q = lax.fori_loop(0, D, d_body, (zeros16, zeros16))

                fo = zeros16
                for f in range(F):
                    fo = fo + lbuf[pl.ds(f * _CHUNK + g * _L, _L)]

                res = 0.5 * (ss - q) + fo
                dst = pl.multiple_of(c * _CHUNK + g * _L, _L)
                out_v[pl.ds(dst, _L)] = res
            return carry

        lax.fori_loop(0, n_chunks, chunk_body, 0)

        out_off = pl.multiple_of(wid * rows_per_w, rows_per_w)
        pltpu.sync_copy(out_v, out_hbm.at[pl.ds(out_off, rows_per_w)])

    return fm_kernel(catT, embT, lr_flat) + bias


def kernel(cat_indices, embed_table, lr_weight, lr_bias):
    B, F = cat_indices.shape
    D = embed_table.shape[1]
    V = embed_table.shape[0] // F
    assert B % (_NW * _CHUNK) == 0 and D == _L and (_CHUNK * F) % 128 == 0

    catT = cat_indices.astype(jnp.int32).T
    embT = embed_table.T
    lr_flat = lr_weight.reshape(-1)
    out = _fm_sc(catT, embT, lr_flat, lr_bias, B=B, F=F, V=V, D=D)
    return out[:, None]


# speedup vs baseline: 3.1400x; 3.1400x over previous
import functools

import jax
import jax.numpy as jnp
from jax import lax
from jax.experimental import pallas as pl
from jax.experimental.pallas import tpu as pltpu
from jax.experimental.pallas import tpu_sc as plsc

_NC = 2
_NS = 16
_NW = _NC * _NS
_L = 16

_CHUNK = 64


@functools.partial(jax.jit, static_argnames=("R", "D"))
def _detile(embT, tail, *, R, D):
    n_tc = R // 128
    rem = R % 128
    out_rows = (R * D) // 128

    full_r = n_tc // _NW
    tail_w = n_tc % _NW
    assert full_r % 2 == 0 and rem % 8 == 0
    loop_r = full_r

    mesh = plsc.VectorSubcoreMesh(core_axis_name="c", subcore_axis_name="s")

    @functools.partial(
        pl.kernel,
        out_type=jax.ShapeDtypeStruct((out_rows, 128), jnp.float32),
        mesh=mesh,
        compiler_params=pltpu.CompilerParams(needs_layout_passes=False),
        scratch_types=[
            pltpu.VMEM((_L, 128), jnp.float32),
            pltpu.VMEM((_L, 128), jnp.float32),
            pltpu.VMEM((_L, 128), jnp.float32),
            pltpu.VMEM((_L, 128), jnp.float32),
            pltpu.SemaphoreType.DMA,
            pltpu.SemaphoreType.DMA,
            pltpu.SemaphoreType.DMA,
            pltpu.SemaphoreType.DMA,
        ],
    )
    def k1(embT_hbm, tail_hbm, out_hbm, in0, in1, ot0, ot1, si0, si1, so0, so1):
        wid = lax.axis_index("s") * _NC + lax.axis_index("c")
        iota16 = lax.iota(jnp.int32, _L)
        ins = (in0, in1)
        ots = (ot0, ot1)
        sis = (si0, si1)
        sos = (so0, so1)

        def issue_in(r, b):
            tc = r * _NW + wid
            col = pl.multiple_of(tc * 128, 128)
            pltpu.async_copy(embT_hbm.at[pl.ds(0, 8), pl.ds(col, 128)],
                             ins[b].at[pl.ds(0, 8)], sis[b])
            pltpu.async_copy(embT_hbm.at[pl.ds(8, 8), pl.ds(col, 128)],
                             ins[b].at[pl.ds(8, 8)], sis[b])

        def wait_in(b):
            pltpu.make_async_copy(embT_hbm.at[pl.ds(0, 8), pl.ds(0, 128)],
                                  ins[b].at[pl.ds(0, 8)], sis[b]).wait()
            pltpu.make_async_copy(embT_hbm.at[pl.ds(0, 8), pl.ds(0, 128)],
                                  ins[b].at[pl.ds(8, 8)], sis[b]).wait()

        def transpose(b, ncols):
            for c in range(ncols):
                vec = plsc.load_gather(
                    ins[b], [iota16, jnp.full((_L,), c, jnp.int32)])
                ots[b][c // 8, pl.ds((c % 8) * _L, _L)] = vec

        def issue_out(r, b):
            tc = r * _NW + wid
            row = pl.multiple_of(tc * 16, 16)
            pltpu.async_copy(ots[b], out_hbm.at[pl.ds(row, 16)], sos[b])

        def wait_out(b):
            pltpu.make_async_copy(embT_hbm.at[pl.ds(0, 16), pl.ds(0, 128)],
                                  ots[b], sos[b]).wait()

        issue_in(0, 0)

        def body(rr, carry):
            for b in (0, 1):
                r = rr * 2 + b
                wait_in(b)

                @pl.when((r + 1 < full_r) | (wid < tail_w))
                def _():
                    issue_in(r + 1, 1 - b)

                transpose(b, 128)

                @pl.when(r >= 2)
                def _():
                    wait_out(b)

                issue_out(r, b)
            return carry

        lax.fori_loop(0, loop_r // 2, body, 0)

        wait_out(0)
        wait_out(1)

        @pl.when(wid < tail_w)
        def _():
            wait_in(0)
            transpose(0, 128)
            issue_out(full_r, 0)
            wait_out(0)

        if rem:
            @pl.when(wid == _NW - 1)
            def _():
                pltpu.sync_copy(tail_hbm, ot0.at[pl.ds(0, rem * D // 128)])
                pltpu.sync_copy(ot0.at[pl.ds(0, rem * D // 128)],
                                out_hbm.at[pl.ds(n_tc * 16, rem * D // 128)])

    return k1(embT, tail)


@functools.partial(jax.jit, static_argnames=("B", "F", "V", "D"))
def _fm_sc(cat_flat, emb, lr_flat, bias, *, B, F, V, D):
    rows_per_w = B // _NW
    n_chunks = rows_per_w // _CHUNK
    idx_len = _CHUNK * F

    mesh = plsc.VectorSubcoreMesh(core_axis_name="c", subcore_axis_name="s")

    @functools.partial(
        pl.kernel,
        out_type=jax.ShapeDtypeStruct((B,), jnp.float32),
        mesh=mesh,
        compiler_params=pltpu.CompilerParams(needs_layout_passes=False,
                                             use_tc_tiling_on_sc=False),
        scratch_types=[
            pltpu.VMEM((idx_len,), jnp.int32),
            pltpu.VMEM((F, _CHUNK), jnp.int32),
            pltpu.VMEM((F * _CHUNK, D), jnp.float32),
            pltpu.VMEM((F, _CHUNK), jnp.float32),
            pltpu.VMEM((B // _NW,), jnp.float32),
            pltpu.SemaphoreType.DMA,
        ],
    )
    def fm_kernel(cat_hbm, emb_hbm, lr_hbm, out_hbm,
                  idxraw_v, idx_v, ebuf, lbuf, out_v, sem):
        wid = lax.axis_index("s") * _NC + lax.axis_index("c")
        w_base = wid * (rows_per_w * F)

        zeros16 = jnp.zeros((_L,), jnp.float32)

        jlane = lax.iota(jnp.int32, _L)
        jF = jlane * F

        def chunk_body(c, carry):
            src_off = pl.multiple_of(w_base + c * idx_len, idx_len)
            pltpu.sync_copy(cat_hbm.at[pl.ds(src_off, idx_len)], idxraw_v)

            for f in range(F):
                for g in range(_CHUNK // _L):
                    addr = jF + (g * _L * F + f)
                    vals = plsc.load_gather(idxraw_v, [addr])
                    idx_v[f, pl.ds(g * _L, _L)] = vals + (f * V)

            copies = []
            for f in range(F):
                copies.append(pltpu.async_copy(
                    emb_hbm.at[idx_v.at[f]],
                    ebuf.at[pl.ds(f * _CHUNK, _CHUNK)], sem))
                copies.append(pltpu.async_copy(
                    lr_hbm.at[idx_v.at[f]], lbuf.at[f], sem))
            for cp in copies:
                cp.wait()

            for g in range(_CHUNK // _L):
                jrow = jlane + (g * _L)
                rowv = [jrow + f * _CHUNK for f in range(F)]

                def d_body(d, acc):
                    ss, q = acc
                    dcol = jnp.broadcast_to(d, (_L,))
                    t = zeros16
                    for f in range(F):
                        e = plsc.load_gather(ebuf, [rowv[f], dcol])
                        t = t + e
                        q = q + e * e
                    return ss + t * t, q

                ss, q = lax.fori_loop(0, D, d_body, (zeros16, zeros16))

                fo = zeros16
                for f in range(F):
                    fo = fo + lbuf[f, pl.ds(g * _L, _L)]

                res = 0.5 * (ss - q) + fo
                dst = pl.multiple_of(c * _CHUNK + g * _L, _L)
                out_v[pl.ds(dst, _L)] = res
            return carry

        lax.fori_loop(0, n_chunks, chunk_body, 0)

        out_off = pl.multiple_of(wid * rows_per_w, rows_per_w)
        pltpu.sync_copy(out_v, out_hbm.at[pl.ds(out_off, rows_per_w)])

    return fm_kernel(cat_flat, emb, lr_flat) + bias


def kernel(cat_indices, embed_table, lr_weight, lr_bias):
    B, F = cat_indices.shape
    D = embed_table.shape[1]
    V = embed_table.shape[0] // F
    R = F * V
    assert B % (_NW * _CHUNK) == 0 and D == _L
    assert R % 8 == 0 and (R % 128) in (0, 64)

    rem = R % 128
    tail = embed_table[R - rem:, :].reshape(rem * D // 128, 128)
    tab = _detile(embed_table.T, tail, R=R, D=D).reshape(R, D)
    cat_flat = cat_indices.astype(jnp.int32).reshape(B * F)
    lr_flat = lr_weight.reshape(-1)
    out = _fm_sc(cat_flat, tab, lr_flat, lr_bias, B=B, F=F, V=V, D=D)
    return out[:, None]


# speedup vs baseline: 5.3022x; 1.6886x over previous
import functools

import jax
import jax.numpy as jnp
from jax import lax
from jax.experimental import pallas as pl
from jax.experimental.pallas import tpu as pltpu
from jax.experimental.pallas import tpu_sc as plsc

_NC = 2
_NS = 16
_NW = _NC * _NS
_L = 16

_CHUNK = 64


@functools.partial(jax.jit, static_argnames=("R", "D"))
def _detile(embT, tail, *, R, D):
    n_tc = R // 128
    rem = R % 128
    out_rows = (R * D) // 128

    full_r = n_tc // _NW
    tail_w = n_tc % _NW
    assert full_r % 2 == 0 and rem % 8 == 0
    loop_r = full_r

    mesh = plsc.VectorSubcoreMesh(core_axis_name="c", subcore_axis_name="s")

    @functools.partial(
        pl.kernel,
        out_type=jax.ShapeDtypeStruct((out_rows, 128), jnp.float32),
        mesh=mesh,
        compiler_params=pltpu.CompilerParams(needs_layout_passes=False),
        scratch_types=[
            pltpu.VMEM((_L, 128), jnp.float32),
            pltpu.VMEM((_L, 128), jnp.float32),
            pltpu.VMEM((_L, 128), jnp.float32),
            pltpu.VMEM((_L, 128), jnp.float32),
            pltpu.SemaphoreType.DMA,
            pltpu.SemaphoreType.DMA,
            pltpu.SemaphoreType.DMA,
            pltpu.SemaphoreType.DMA,
        ],
    )
    def k1(embT_hbm, tail_hbm, out_hbm, in0, in1, ot0, ot1, si0, si1, so0, so1):
        wid = lax.axis_index("s") * _NC + lax.axis_index("c")
        iota16 = lax.iota(jnp.int32, _L)
        ins = (in0, in1)
        ots = (ot0, ot1)
        sis = (si0, si1)
        sos = (so0, so1)

        def issue_in(r, b):
            tc = r * _NW + wid
            col = pl.multiple_of(tc * 128, 128)
            pltpu.async_copy(embT_hbm.at[pl.ds(0, 8), pl.ds(col, 128)],
                             ins[b].at[pl.ds(0, 8)], sis[b])
            pltpu.async_copy(embT_hbm.at[pl.ds(8, 8), pl.ds(col, 128)],
                             ins[b].at[pl.ds(8, 8)], sis[b])

        def wait_in(b):
            pltpu.make_async_copy(embT_hbm.at[pl.ds(0, 8), pl.ds(0, 128)],
                                  ins[b].at[pl.ds(0, 8)], sis[b]).wait()
            pltpu.make_async_copy(embT_hbm.at[pl.ds(0, 8), pl.ds(0, 128)],
                                  ins[b].at[pl.ds(8, 8)], sis[b]).wait()

        def transpose(b, ncols):
            for c0 in range(0, ncols, 32):
                cs = range(c0, min(c0 + 32, ncols))
                vecs = [plsc.load_gather(
                    ins[b], [iota16, jnp.full((_L,), c, jnp.int32)])
                    for c in cs]
                for c, vec in zip(cs, vecs):
                    ots[b][c // 8, pl.ds((c % 8) * _L, _L)] = vec

        def issue_out(r, b):
            tc = r * _NW + wid
            row = pl.multiple_of(tc * 16, 16)
            pltpu.async_copy(ots[b], out_hbm.at[pl.ds(row, 16)], sos[b])

        def wait_out(b):
            pltpu.make_async_copy(embT_hbm.at[pl.ds(0, 16), pl.ds(0, 128)],
                                  ots[b], sos[b]).wait()

        issue_in(0, 0)

        def body(rr, carry):
            for b in (0, 1):
                r = rr * 2 + b
                wait_in(b)

                @pl.when((r + 1 < full_r) | (wid < tail_w))
                def _():
                    issue_in(r + 1, 1 - b)

                transpose(b, 128)

                @pl.when(r >= 2)
                def _():
                    wait_out(b)

                issue_out(r, b)
            return carry

        lax.fori_loop(0, loop_r // 2, body, 0)

        wait_out(0)
        wait_out(1)

        @pl.when(wid < tail_w)
        def _():
            wait_in(0)
            transpose(0, 128)
            issue_out(full_r, 0)
            wait_out(0)

        if rem:
            @pl.when(wid == _NW - 1)
            def _():
                pltpu.sync_copy(tail_hbm, ot0.at[pl.ds(0, rem * D // 128)])
                pltpu.sync_copy(ot0.at[pl.ds(0, rem * D // 128)],
                                out_hbm.at[pl.ds(n_tc * 16, rem * D // 128)])

    return k1(embT, tail)


@functools.partial(jax.jit, static_argnames=("B", "F", "V", "D"))
def _fm_sc(cat_flat, emb, lr_flat, bias, *, B, F, V, D):
    rows_per_w = B // _NW
    n_chunks = rows_per_w // _CHUNK
    idx_len = _CHUNK * F

    mesh = plsc.VectorSubcoreMesh(core_axis_name="c", subcore_axis_name="s")

    @functools.partial(
        pl.kernel,
        out_type=jax.ShapeDtypeStruct((B,), jnp.float32),
        mesh=mesh,
        compiler_params=pltpu.CompilerParams(needs_layout_passes=False,
                                             use_tc_tiling_on_sc=False),
        scratch_types=[
            pltpu.VMEM((idx_len,), jnp.int32),
            pltpu.VMEM((F, _CHUNK), jnp.int32),
            pltpu.VMEM((F * _CHUNK, D), jnp.float32),
            pltpu.VMEM((F, _CHUNK), jnp.float32),
            pltpu.VMEM((B // _NW,), jnp.float32),
            pltpu.SemaphoreType.DMA,
        ],
    )
    def fm_kernel(cat_hbm, emb_hbm, lr_hbm, out_hbm,
                  idxraw_v, idx_v, ebuf, lbuf, out_v, sem):
        wid = lax.axis_index("s") * _NC + lax.axis_index("c")
        w_base = wid * (rows_per_w * F)

        zeros16 = jnp.zeros((_L,), jnp.float32)

        jlane = lax.iota(jnp.int32, _L)
        jF = jlane * F

        def chunk_body(c, carry):
            src_off = pl.multiple_of(w_base + c * idx_len, idx_len)
            pltpu.sync_copy(cat_hbm.at[pl.ds(src_off, idx_len)], idxraw_v)

            for f in range(F):
                for g in range(_CHUNK // _L):
                    addr = jF + (g * _L * F + f)
                    vals = plsc.load_gather(idxraw_v, [addr])
                    idx_v[f, pl.ds(g * _L, _L)] = vals + (f * V)

            copies = []
            for f in range(F):
                copies.append(pltpu.async_copy(
                    emb_hbm.at[idx_v.at[f]],
                    ebuf.at[pl.ds(f * _CHUNK, _CHUNK)], sem))
                copies.append(pltpu.async_copy(
                    lr_hbm.at[idx_v.at[f]], lbuf.at[f], sem))
            for cp in copies:
                cp.wait()

            for g in range(_CHUNK // _L):
                jrow = jlane + (g * _L)
                rowv = [jrow + f * _CHUNK for f in range(F)]

                def d_body(d, acc):
                    ss, q = acc
                    dcol = jnp.broadcast_to(d, (_L,))
                    t = zeros16
                    for f in range(F):
                        e = plsc.load_gather(ebuf, [rowv[f], dcol])
                        t = t + e
                        q = q + e * e
                    return ss + t * t, q

                ss, q = lax.fori_loop(0, D, d_body, (zeros16, zeros16))

                fo = zeros16
                for f in range(F):
                    fo = fo + lbuf[f, pl.ds(g * _L, _L)]

                res = 0.5 * (ss - q) + fo
                dst = pl.multiple_of(c * _CHUNK + g * _L, _L)
                out_v[pl.ds(dst, _L)] = res
            return carry

        lax.fori_loop(0, n_chunks, chunk_body, 0)

        out_off = pl.multiple_of(wid * rows_per_w, rows_per_w)
        pltpu.sync_copy(out_v, out_hbm.at[pl.ds(out_off, rows_per_w)])

    return fm_kernel(cat_flat, emb, lr_flat) + bias


def kernel(cat_indices, embed_table, lr_weight, lr_bias):
    B, F = cat_indices.shape
    D = embed_table.shape[1]
    V = embed_table.shape[0] // F
    R = F * V
    assert B % (_NW * _CHUNK) == 0 and D == _L
    assert R % 8 == 0 and (R % 128) in (0, 64)

    rem = R % 128
    tail = embed_table[R - rem:, :].reshape(rem * D // 128, 128)
    tab = _detile(embed_table.T, tail, R=R, D=D).reshape(R, D)
    cat_flat = cat_indices.astype(jnp.int32).reshape(B * F)
    lr_flat = lr_weight.reshape(-1)
    out = _fm_sc(cat_flat, tab, lr_flat, lr_bias, B=B, F=F, V=V, D=D)
    return out[:, None]
